# baseline (speedup 1.0000x reference)
import functools

import jax
import jax.numpy as jnp
from jax import lax
from jax.experimental import pallas as pl
from jax.experimental.pallas import tpu as pltpu
from jax.experimental.pallas import tpu_sc as plsc

BINS = 100000
EMB = 16
NF = 26
BATCH = 4096
N_DENSE = 13
HASH_MULT = 2654435761

NC = 2
NS = 16
NW = NC * NS

GCHUNK = 1024
VIEW_LEN = (NF * EMB - EMB + 1) * BINS


def _emb_gather(sparse_flat, table_flat, rows_w):
    vecs = rows_w // 16
    elems_w = rows_w * EMB
    nch = rows_w // GCHUNK
    tail = rows_w - nch * GCHUNK
    mesh = plsc.VectorSubcoreMesh(core_axis_name="c", subcore_axis_name="s")

    @functools.partial(
        pl.kernel,
        mesh=mesh,
        out_type=jax.ShapeDtypeStruct((NW * elems_w,), jnp.float32),
        scratch_types=[
            pltpu.VMEM((rows_w,), jnp.int32),
            pltpu.VMEM((rows_w,), jnp.int32),
            pltpu.VMEM((elems_w,), jnp.float32),
            pltpu.SemaphoreType.DMA,
        ],
        compiler_params=pltpu.CompilerParams(use_tc_tiling_on_sc=False),
    )
    def k(idx_hbm, table_hbm, out_hbm, idx_v, base_v, vals_v, sem):
        wid = lax.axis_index("s") * NC + lax.axis_index("c")
        base = wid * rows_w
        pltpu.sync_copy(idx_hbm.at[pl.ds(base, rows_w)], idx_v)
        lane = lax.iota(jnp.int32, 16)

        def hash_body(i, carry):
            v = idx_v[pl.ds(i * 16, 16)]
            h = (v.astype(jnp.uint32) * jnp.uint32(HASH_MULT)) % jnp.uint32(BINS)
            f = (i * 16 + lane) % NF
            base_v[pl.ds(i * 16, 16)] = h.astype(jnp.int32) + f * (EMB * BINS)
            return carry

        lax.fori_loop(0, vecs, hash_body, 0)

        def fire_chunk(c, n):
            idx_ref = base_v.at[pl.ds(c * GCHUNK, n)]
            copies = []
            for d in range(EMB):
                copies.append(pltpu.async_copy(
                    table_hbm.at[pl.ds(d * BINS, VIEW_LEN)].at[idx_ref],
                    vals_v.at[pl.ds(d * rows_w + c * GCHUNK, n)],
                    sem,
                ))
            for cp in copies:
                cp.wait()

        def fire_body(c, carry):
            fire_chunk(c, GCHUNK)
            return carry

        if nch:
            lax.fori_loop(0, nch, fire_body, 0)
        if tail:
            fire_chunk(nch, tail)
        pltpu.sync_copy(vals_v, out_hbm.at[pl.ds(wid * elems_w, elems_w)])

    return k(sparse_flat, table_flat)


def _mlp_body(xd, xe, w1d, w1e, b1, w2, b2, w3, b3, wo, bo, out):
    f32 = jnp.float32
    h = (
        jnp.dot(xd[...], w1d[...], preferred_element_type=f32)
        + jnp.dot(xe[...], w1e[...], preferred_element_type=f32)
        + b1[...]
    )
    h = jnp.maximum(h, 0.0)
    h = jnp.maximum(jnp.dot(h, w2[...], preferred_element_type=f32) + b2[...], 0.0)
    h = jnp.maximum(jnp.dot(h, w3[...], preferred_element_type=f32) + b3[...], 0.0)
    z = jnp.dot(h, wo[...], preferred_element_type=f32) + bo[...]
    out[...] = jax.nn.sigmoid(z)


def _mlp(dense, embs, w1d, w1e, b1, w2, b2, w3, b3, wo, bo):
    BB = 512
    batch = dense.shape[0]
    grid = batch // BB
    full = lambda i: (0, 0)
    return pl.pallas_call(
        _mlp_body,
        grid=(grid,),
        in_specs=[
            pl.BlockSpec((BB, N_DENSE), lambda i: (i, 0)),
            pl.BlockSpec((BB, NF * EMB), lambda i: (i, 0)),
            pl.BlockSpec((N_DENSE, 1024), full),
            pl.BlockSpec((NF * EMB, 1024), full),
            pl.BlockSpec((1, 1024), full),
            pl.BlockSpec((1024, 512), full),
            pl.BlockSpec((1, 512), full),
            pl.BlockSpec((512, 256), full),
            pl.BlockSpec((1, 256), full),
            pl.BlockSpec((256, 1), full),
            pl.BlockSpec((1, 1), full),
        ],
        out_specs=pl.BlockSpec((BB, 1), lambda i: (i, 0)),
        out_shape=jax.ShapeDtypeStruct((batch, 1), jnp.float32),
    )(dense, embs, w1d, w1e, b1, w2, b2, w3, b3, wo, bo)


NH = 2


def kernel(dense, sparse_idx, emb_tables, W1, b1, W2, b2, W3, b3, Wo, bo):
    table_flat = jnp.transpose(emb_tables, (0, 2, 1)).reshape(-1)
    bh = BATCH // NH
    rows_w = bh * NF // NW
    outs = []
    for s in range(NH):
        sparse_flat = sparse_idx[s * bh:(s + 1) * bh].reshape(-1)
        gathered = _emb_gather(sparse_flat, table_flat, rows_w)
        embs = (
            gathered.reshape(NW, EMB, rows_w)
            .transpose(0, 2, 1)
            .reshape(bh, NF * EMB)
        )
        outs.append(_mlp(
            dense[s * bh:(s + 1) * bh], embs,
            W1[:N_DENSE], W1[N_DENSE:], b1.reshape(1, -1),
            W2, b2.reshape(1, -1),
            W3, b3.reshape(1, -1),
            Wo, bo.reshape(1, -1),
        ))
    return jnp.concatenate(outs, axis=0)


# speedup vs baseline: 1.0742x; 1.0742x over previous
import functools

import jax
import jax.numpy as jnp
from jax import lax
from jax.experimental import pallas as pl
from jax.experimental.pallas import tpu as pltpu
from jax.experimental.pallas import tpu_sc as plsc

BINS = 100000
EMB = 16
NF = 26
BATCH = 4096
N_DENSE = 13
HASH_MULT = 2654435761

NC = 2
NS = 16
NW = NC * NS
ROWS_W = BATCH * NF // NW
VECS = ROWS_W // 16
ELEMS_W = ROWS_W * EMB

GCHUNK = 3328
NCH = ROWS_W // GCHUNK
TAIL = ROWS_W - NCH * GCHUNK
VIEW_LEN = (NF * EMB - EMB + 1) * BINS


def _emb_gather(sparse_flat, table_flat):
    mesh = plsc.VectorSubcoreMesh(core_axis_name="c", subcore_axis_name="s")

    @functools.partial(
        pl.kernel,
        mesh=mesh,
        out_type=jax.ShapeDtypeStruct((NW * ELEMS_W,), jnp.float32),
        scratch_types=[
            pltpu.VMEM((ROWS_W,), jnp.int32),
            pltpu.VMEM((ROWS_W,), jnp.int32),
            pltpu.VMEM((ELEMS_W,), jnp.float32),
            pltpu.SemaphoreType.DMA,
        ],
        compiler_params=pltpu.CompilerParams(use_tc_tiling_on_sc=False),
    )
    def k(idx_hbm, table_hbm, out_hbm, idx_v, base_v, vals_v, sem):
        wid = lax.axis_index("s") * NC + lax.axis_index("c")
        base = wid * ROWS_W
        pltpu.sync_copy(idx_hbm.at[pl.ds(base, ROWS_W)], idx_v)
        lane = lax.iota(jnp.int32, 16)

        def hash_body(i, carry):
            v = idx_v[pl.ds(i * 16, 16)]
            h = (v.astype(jnp.uint32) * jnp.uint32(HASH_MULT)) % jnp.uint32(BINS)
            f = (i * 16 + lane) % NF
            base_v[pl.ds(i * 16, 16)] = h.astype(jnp.int32) + f * (EMB * BINS)
            return carry

        lax.fori_loop(0, VECS, hash_body, 0)

        def fire_chunk(c, n):
            idx_ref = base_v.at[pl.ds(c * GCHUNK, n)]
            copies = []
            for d in range(EMB):
                copies.append(pltpu.async_copy(
                    table_hbm.at[pl.ds(d * BINS, VIEW_LEN)].at[idx_ref],
                    vals_v.at[pl.ds(d * ROWS_W + c * GCHUNK, n)],
                    sem,
                ))
            for cp in copies:
                cp.wait()

        def fire_body(c, carry):
            fire_chunk(c, GCHUNK)
            return carry

        lax.fori_loop(0, NCH, fire_body, 0)
        if TAIL:
            fire_chunk(NCH, TAIL)
        pltpu.sync_copy(vals_v, out_hbm.at[pl.ds(wid * ELEMS_W, ELEMS_W)])

    return k(sparse_flat, table_flat)


def _mlp_body(xd, xe, w1d, w1e, b1, w2, b2, w3, b3, wo, bo, out):
    f32 = jnp.float32
    h = (
        jnp.dot(xd[...], w1d[...], preferred_element_type=f32)
        + jnp.dot(xe[...], w1e[...], preferred_element_type=f32)
        + b1[...]
    )
    h = jnp.maximum(h, 0.0)
    h = jnp.maximum(jnp.dot(h, w2[...], preferred_element_type=f32) + b2[...], 0.0)
    h = jnp.maximum(jnp.dot(h, w3[...], preferred_element_type=f32) + b3[...], 0.0)
    z = jnp.dot(h, wo[...], preferred_element_type=f32) + bo[...]
    out[...] = jax.nn.sigmoid(z)


def _mlp(dense, embs, w1d, w1e, b1, w2, b2, w3, b3, wo, bo):
    BB = 512
    grid = BATCH // BB
    full = lambda i: (0, 0)
    return pl.pallas_call(
        _mlp_body,
        grid=(grid,),
        in_specs=[
            pl.BlockSpec((BB, N_DENSE), lambda i: (i, 0)),
            pl.BlockSpec((BB, NF * EMB), lambda i: (i, 0)),
            pl.BlockSpec((N_DENSE, 1024), full),
            pl.BlockSpec((NF * EMB, 1024), full),
            pl.BlockSpec((1, 1024), full),
            pl.BlockSpec((1024, 512), full),
            pl.BlockSpec((1, 512), full),
            pl.BlockSpec((512, 256), full),
            pl.BlockSpec((1, 256), full),
            pl.BlockSpec((256, 1), full),
            pl.BlockSpec((1, 1), full),
        ],
        out_specs=pl.BlockSpec((BB, 1), lambda i: (i, 0)),
        out_shape=jax.ShapeDtypeStruct((BATCH, 1), jnp.float32),
    )(dense, embs, w1d, w1e, b1, w2, b2, w3, b3, wo, bo)


def kernel(dense, sparse_idx, emb_tables, W1, b1, W2, b2, W3, b3, Wo, bo):
    table_flat = jnp.transpose(emb_tables, (0, 2, 1)).reshape(-1)
    sparse_flat = sparse_idx.reshape(-1)
    gathered = _emb_gather(sparse_flat, table_flat)
    embs = (
        gathered.reshape(NW, EMB, ROWS_W)
        .transpose(0, 2, 1)
        .reshape(BATCH, NF * EMB)
    )
    return _mlp(
        dense, embs,
        W1[:N_DENSE], W1[N_DENSE:], b1.reshape(1, -1),
        W2, b2.reshape(1, -1),
        W3, b3.reshape(1, -1),
        Wo, bo.reshape(1, -1),
    )
